# baseline (device time: 88783 ns/iter reference)
import jax
import jax.numpy as jnp
from jax import lax
from jax.experimental import pallas as pl
from jax.experimental.pallas import tpu as pltpu

N_DEV = 16


def kernel(x, router_W, route_idx, expert_W):
    T, D = x.shape
    E_LOC, _, H = expert_W.shape
    T_LOC = T // N_DEV
    N_HOPS = N_DEV - 1

    def body(x_ref, rw_ref, idx_ref, ew_ref, out_ref,
             w_ref, ew_bf_ref, send_ref, recv_ref, send_sem, recv_sems):
        my = lax.axis_index("i")
        left = (my - 1) % N_DEV
        right = (my + 1) % N_DEV

        barrier_sem = pltpu.get_barrier_semaphore()
        for nbr in (left, right):
            pl.semaphore_signal(
                barrier_sem, inc=1,
                device_id=(nbr,), device_id_type=pl.DeviceIdType.MESH,
            )
        pl.semaphore_wait(barrier_sem, 2)

        scores = lax.dot(x_ref[:, :], rw_ref[:, :],
                         preferred_element_type=jnp.float32)
        n_exp = scores.shape[1]
        e0 = idx_ref[:, 0:1]
        e1 = idx_ref[:, 1:2]
        eiota = lax.broadcasted_iota(jnp.int32, (T, n_exp), 1)
        s0 = jnp.sum(jnp.where(eiota == e0, scores, 0.0), axis=1,
                     keepdims=True)
        s1 = jnp.sum(jnp.where(eiota == e1, scores, 0.0), axis=1,
                     keepdims=True)
        w0 = jax.nn.sigmoid(s0 - s1)
        w1 = 1.0 - w0
        gids = my * E_LOC + lax.broadcasted_iota(jnp.int32, (T, E_LOC), 1)
        w_ref[:, :] = (jnp.where(e0 == gids, w0, 0.0)
                       + jnp.where(e1 == gids, w1, 0.0))

        for le in range(E_LOC):
            ew_bf_ref[le, :, :] = ew_ref[le, :, :].astype(jnp.bfloat16)

        def compute_chunk(c):
            start = c * T_LOC
            xs = x_ref[pl.ds(start, T_LOC), :]
            wc = w_ref[pl.ds(start, T_LOC), :]
            acc = jnp.zeros((T_LOC, H), jnp.float32)
            for le in range(E_LOC):
                xw = (xs * wc[:, le:le + 1]).astype(jnp.bfloat16)
                acc = acc + lax.dot(xw, ew_bf_ref[le, :, :],
                                    preferred_element_type=jnp.float32)
            return acc

        send_ref[:, :] = compute_chunk((my - 1) % N_DEV).astype(jnp.bfloat16)
        for s in range(N_HOPS):
            rdma = pltpu.make_async_remote_copy(
                src_ref=send_ref,
                dst_ref=recv_ref.at[s],
                send_sem=send_sem,
                recv_sem=recv_sems.at[s],
                device_id=(right,),
                device_id_type=pl.DeviceIdType.MESH,
            )
            rdma.start()
            nxt = compute_chunk((my - s - 2) % N_DEV)
            rdma.wait_send()
            rdma.wait_recv()
            if s < N_HOPS - 1:
                send_ref[:, :] = (recv_ref[s, :, :].astype(jnp.float32)
                                  + nxt).astype(jnp.bfloat16)
            else:
                out_ref[:, :] = recv_ref[s, :, :].astype(jnp.float32) + nxt

    return pl.pallas_call(
        body,
        out_shape=jax.ShapeDtypeStruct((T_LOC, H), jnp.float32),
        in_specs=[
            pl.BlockSpec(memory_space=pltpu.VMEM),
            pl.BlockSpec(memory_space=pltpu.VMEM),
            pl.BlockSpec(memory_space=pltpu.VMEM),
            pl.BlockSpec(memory_space=pltpu.VMEM),
        ],
        out_specs=pl.BlockSpec(memory_space=pltpu.VMEM),
        scratch_shapes=[
            pltpu.VMEM((T, E_LOC), jnp.float32),
            pltpu.VMEM((E_LOC, D, H), jnp.bfloat16),
            pltpu.VMEM((T_LOC, H), jnp.bfloat16),
            pltpu.VMEM((N_HOPS, T_LOC, H), jnp.bfloat16),
            pltpu.SemaphoreType.DMA,
            pltpu.SemaphoreType.DMA((N_HOPS,)),
        ],
        compiler_params=pltpu.CompilerParams(collective_id=0),
    )(x, router_W, route_idx, expert_W)


# device time: 60902 ns/iter; 1.4578x vs baseline; 1.4578x over previous
import jax
import jax.numpy as jnp
from jax import lax
from jax.experimental import pallas as pl
from jax.experimental.pallas import tpu as pltpu

N_DEV = 16


def kernel(x, router_W, route_idx, expert_W):
    T, D = x.shape
    E_LOC, _, H = expert_W.shape
    T_LOC = T // N_DEV
    N_HOPS = N_DEV - 1

    def body(x_ref, rw_ref, idx_ref, ew_ref, out_ref,
             w_ref, ew_bf_ref, send_r_ref, send_l_ref, recv_r_ref,
             recv_l_ref, send_r_sem, send_l_sem, recv_r_sems, recv_l_sems):
        my = lax.axis_index("i")
        left = (my - 1) % N_DEV
        right = (my + 1) % N_DEV

        barrier_sem = pltpu.get_barrier_semaphore()
        for nbr in (left, right):
            pl.semaphore_signal(
                barrier_sem, inc=1,
                device_id=(nbr,), device_id_type=pl.DeviceIdType.MESH,
            )
        pl.semaphore_wait(barrier_sem, 2)

        scores = lax.dot(x_ref[:, :], rw_ref[:, :],
                         preferred_element_type=jnp.float32)
        n_exp = scores.shape[1]
        e0 = idx_ref[:, 0:1]
        e1 = idx_ref[:, 1:2]
        eiota = lax.broadcasted_iota(jnp.int32, (T, n_exp), 1)
        s0 = jnp.sum(jnp.where(eiota == e0, scores, 0.0), axis=1,
                     keepdims=True)
        s1 = jnp.sum(jnp.where(eiota == e1, scores, 0.0), axis=1,
                     keepdims=True)
        w0 = jax.nn.sigmoid(s0 - s1)
        w1 = 1.0 - w0
        gids = my * E_LOC + lax.broadcasted_iota(jnp.int32, (T, E_LOC), 1)
        w_ref[:, :] = (jnp.where(e0 == gids, w0, 0.0)
                       + jnp.where(e1 == gids, w1, 0.0))

        for le in range(E_LOC):
            ew_bf_ref[le * D:(le + 1) * D, :] = (
                ew_ref[le, :, :].astype(jnp.bfloat16))

        def _moe_partial(xs, wc):
            parts = [(xs * wc[:, le:le + 1]).astype(jnp.bfloat16)
                     for le in range(E_LOC)]
            xt = jnp.concatenate(parts, axis=1)
            return lax.dot(xt, ew_bf_ref[:, :],
                           preferred_element_type=jnp.float32)

        def compute_chunk(c):
            start = c * T_LOC
            return _moe_partial(x_ref[pl.ds(start, T_LOC), :],
                                w_ref[pl.ds(start, T_LOC), :])

        def compute_chunk_pair(ca, cb):
            sa, sb = ca * T_LOC, cb * T_LOC
            xs = jnp.concatenate(
                [x_ref[pl.ds(sa, T_LOC), :], x_ref[pl.ds(sb, T_LOC), :]], 0)
            wc = jnp.concatenate(
                [w_ref[pl.ds(sa, T_LOC), :], w_ref[pl.ds(sb, T_LOC), :]], 0)
            acc = _moe_partial(xs, wc)
            return acc[:T_LOC, :], acc[T_LOC:, :]

        p_r0, p_l0 = compute_chunk_pair((my + 7) % N_DEV, (my + 8) % N_DEV)
        send_r_ref[:, :] = p_r0.astype(jnp.bfloat16)
        send_l_ref[:, :] = p_l0.astype(jnp.bfloat16)
        own = None
        for s in range(8):
            if s < 7:
                rdma_r = pltpu.make_async_remote_copy(
                    src_ref=send_r_ref,
                    dst_ref=recv_r_ref.at[s],
                    send_sem=send_r_sem,
                    recv_sem=recv_r_sems.at[s],
                    device_id=(right,),
                    device_id_type=pl.DeviceIdType.MESH,
                )
                rdma_r.start()
            rdma_l = pltpu.make_async_remote_copy(
                src_ref=send_l_ref,
                dst_ref=recv_l_ref.at[s],
                send_sem=send_l_sem,
                recv_sem=recv_l_sems.at[s],
                device_id=(left,),
                device_id_type=pl.DeviceIdType.MESH,
            )
            rdma_l.start()
            nxt_r = nxt_l = None
            if s < 6:
                nxt_r, nxt_l = compute_chunk_pair((my + 6 - s) % N_DEV,
                                                  (my - 7 + s) % N_DEV)
            elif s == 6:
                nxt_l = compute_chunk((my - 1) % N_DEV)
            else:
                own = compute_chunk(my)
            if s < 7:
                rdma_r.wait_send()
                rdma_r.wait_recv()
                if s < 6:
                    send_r_ref[:, :] = (
                        recv_r_ref[s, :, :].astype(jnp.float32) + nxt_r
                    ).astype(jnp.bfloat16)
            rdma_l.wait_send()
            rdma_l.wait_recv()
            if s < 7:
                send_l_ref[:, :] = (
                    recv_l_ref[s, :, :].astype(jnp.float32) + nxt_l
                ).astype(jnp.bfloat16)
        out_ref[:, :] = (own
                         + recv_r_ref[6, :, :].astype(jnp.float32)
                         + recv_l_ref[7, :, :].astype(jnp.float32))

    return pl.pallas_call(
        body,
        out_shape=jax.ShapeDtypeStruct((T_LOC, H), jnp.float32),
        in_specs=[
            pl.BlockSpec(memory_space=pltpu.VMEM),
            pl.BlockSpec(memory_space=pltpu.VMEM),
            pl.BlockSpec(memory_space=pltpu.VMEM),
            pl.BlockSpec(memory_space=pltpu.VMEM),
        ],
        out_specs=pl.BlockSpec(memory_space=pltpu.VMEM),
        scratch_shapes=[
            pltpu.VMEM((T, E_LOC), jnp.float32),
            pltpu.VMEM((E_LOC * D, H), jnp.bfloat16),
            pltpu.VMEM((T_LOC, H), jnp.bfloat16),
            pltpu.VMEM((T_LOC, H), jnp.bfloat16),
            pltpu.VMEM((7, T_LOC, H), jnp.bfloat16),
            pltpu.VMEM((8, T_LOC, H), jnp.bfloat16),
            pltpu.SemaphoreType.DMA,
            pltpu.SemaphoreType.DMA,
            pltpu.SemaphoreType.DMA((7,)),
            pltpu.SemaphoreType.DMA((8,)),
        ],
        compiler_params=pltpu.CompilerParams(collective_id=0),
    )(x, router_W, route_idx, expert_W)


# device time: 58952 ns/iter; 1.5060x vs baseline; 1.0331x over previous
import jax
import jax.numpy as jnp
from jax import lax
from jax.experimental import pallas as pl
from jax.experimental.pallas import tpu as pltpu

N_DEV = 16

PERM = [0, 1, 5, 9, 13, 14, 10, 6, 2, 3, 7, 11, 15, 12, 8, 4]
RINV = [0, 1, 8, 9, 15, 2, 7, 10, 14, 3, 6, 11, 13, 4, 5, 12]


def kernel(x, router_W, route_idx, expert_W):
    T, D = x.shape
    E_LOC, _, H = expert_W.shape
    T_LOC = T // N_DEV
    N_HOPS = N_DEV - 1

    def body(x_ref, rw_ref, idx_ref, ew_ref, perm_ref, rinv_ref, out_ref,
             w_ref, x_bf_ref, ew_bf_ref, send_r_ref, send_l_ref, recv_r_ref,
             recv_l_ref, send_r_sem, send_l_sem, recv_r_sems, recv_l_sems):
        my = lax.axis_index("i")

        idx16 = lax.broadcasted_iota(jnp.int32, (1, N_DEV), 1)
        perm_a = perm_ref[:, :]
        rinv_a = rinv_ref[:, :]

        def perm_at(r):
            return jnp.sum(jnp.where(idx16 == (r % N_DEV), perm_a, 0))

        r_me = jnp.sum(jnp.where(idx16 == my, rinv_a, 0))
        left = perm_at(r_me - 1)
        right = perm_at(r_me + 1)

        barrier_sem = pltpu.get_barrier_semaphore()
        for nbr in (left, right):
            pl.semaphore_signal(
                barrier_sem, inc=1,
                device_id=(nbr,), device_id_type=pl.DeviceIdType.MESH,
            )
        pl.semaphore_wait(barrier_sem, 2)

        scores = lax.dot(x_ref[:, :], rw_ref[:, :],
                         preferred_element_type=jnp.float32)
        n_exp = scores.shape[1]
        e0 = idx_ref[:, 0:1]
        e1 = idx_ref[:, 1:2]
        eiota = lax.broadcasted_iota(jnp.int32, (T, n_exp), 1)
        s0 = jnp.sum(jnp.where(eiota == e0, scores, 0.0), axis=1,
                     keepdims=True)
        s1 = jnp.sum(jnp.where(eiota == e1, scores, 0.0), axis=1,
                     keepdims=True)
        w0 = jax.nn.sigmoid(s0 - s1)
        w1 = 1.0 - w0
        gids = my * E_LOC + lax.broadcasted_iota(jnp.int32, (T, E_LOC), 1)
        w_ref[:, :] = (jnp.where(e0 == gids, w0, 0.0)
                       + jnp.where(e1 == gids, w1, 0.0)
                       ).astype(jnp.bfloat16)
        x_bf_ref[:, :] = x_ref[:, :].astype(jnp.bfloat16)

        for le in range(E_LOC):
            ew_bf_ref[le * D:(le + 1) * D, :] = (
                ew_ref[le, :, :].astype(jnp.bfloat16))

        def _moe_partial(xs, wc):
            parts = [xs * wc[:, le:le + 1] for le in range(E_LOC)]
            xt = jnp.concatenate(parts, axis=1)
            return lax.dot(xt, ew_bf_ref[:, :],
                           preferred_element_type=jnp.float32)

        def compute_chunk(c):
            start = c * T_LOC
            return _moe_partial(x_bf_ref[pl.ds(start, T_LOC), :],
                                w_ref[pl.ds(start, T_LOC), :])

        def compute_chunk_pair(ca, cb):
            sa, sb = ca * T_LOC, cb * T_LOC
            xs = jnp.concatenate(
                [x_bf_ref[pl.ds(sa, T_LOC), :], x_bf_ref[pl.ds(sb, T_LOC), :]],
                0)
            wc = jnp.concatenate(
                [w_ref[pl.ds(sa, T_LOC), :], w_ref[pl.ds(sb, T_LOC), :]], 0)
            acc = _moe_partial(xs, wc)
            return acc[:T_LOC, :], acc[T_LOC:, :]

        p_r0, p_l0 = compute_chunk_pair(perm_at(r_me + 7), perm_at(r_me + 8))
        send_r_ref[:, :] = p_r0.astype(jnp.bfloat16)
        send_l_ref[:, :] = p_l0.astype(jnp.bfloat16)
        own = None
        for s in range(8):
            if s < 7:
                rdma_r = pltpu.make_async_remote_copy(
                    src_ref=send_r_ref,
                    dst_ref=recv_r_ref.at[s],
                    send_sem=send_r_sem,
                    recv_sem=recv_r_sems.at[s],
                    device_id=(right,),
                    device_id_type=pl.DeviceIdType.MESH,
                )
                rdma_r.start()
            rdma_l = pltpu.make_async_remote_copy(
                src_ref=send_l_ref,
                dst_ref=recv_l_ref.at[s],
                send_sem=send_l_sem,
                recv_sem=recv_l_sems.at[s],
                device_id=(left,),
                device_id_type=pl.DeviceIdType.MESH,
            )
            rdma_l.start()
            nxt_r = nxt_l = None
            if s < 6:
                nxt_r, nxt_l = compute_chunk_pair(perm_at(r_me + 6 - s),
                                                  perm_at(r_me - 7 + s))
            elif s == 6:
                nxt_l = compute_chunk(perm_at(r_me - 1))
            else:
                own = compute_chunk(my)
            if s < 7:
                rdma_r.wait_send()
                rdma_r.wait_recv()
                if s < 6:
                    send_r_ref[:, :] = (
                        recv_r_ref[s, :, :].astype(jnp.float32) + nxt_r
                    ).astype(jnp.bfloat16)
            rdma_l.wait_send()
            rdma_l.wait_recv()
            if s < 7:
                send_l_ref[:, :] = (
                    recv_l_ref[s, :, :].astype(jnp.float32) + nxt_l
                ).astype(jnp.bfloat16)
        out_ref[:, :] = (own
                         + recv_r_ref[6, :, :].astype(jnp.float32)
                         + recv_l_ref[7, :, :].astype(jnp.float32))

    return pl.pallas_call(
        body,
        out_shape=jax.ShapeDtypeStruct((T_LOC, H), jnp.float32),
        in_specs=[pl.BlockSpec(memory_space=pltpu.VMEM)] * 6,
        out_specs=pl.BlockSpec(memory_space=pltpu.VMEM),
        scratch_shapes=[
            pltpu.VMEM((T, E_LOC), jnp.bfloat16),
            pltpu.VMEM((T, D), jnp.bfloat16),
            pltpu.VMEM((E_LOC * D, H), jnp.bfloat16),
            pltpu.VMEM((T_LOC, H), jnp.bfloat16),
            pltpu.VMEM((T_LOC, H), jnp.bfloat16),
            pltpu.VMEM((7, T_LOC, H), jnp.bfloat16),
            pltpu.VMEM((8, T_LOC, H), jnp.bfloat16),
            pltpu.SemaphoreType.DMA,
            pltpu.SemaphoreType.DMA,
            pltpu.SemaphoreType.DMA((7,)),
            pltpu.SemaphoreType.DMA((8,)),
        ],
        compiler_params=pltpu.CompilerParams(collective_id=0),
    )(x, router_W, route_idx, expert_W,
      jnp.array([PERM], jnp.int32), jnp.array([RINV], jnp.int32))
